# Initial kernel scaffold; baseline (speedup 1.0000x reference)
import jax
import jax.numpy as jnp
from jax.experimental import pallas as pl


def kernel(x, edge_index, edge_attr, edge_embed, W_pre, b_pre, W_msg0, b_msg0, W_msg1, b_msg1, W_msg2, b_msg2, W_msg3, b_msg3, W_r0, b_r0, W_r1, b_r1, W_r2, b_r2, W_post, b_post, init0, emb0, emb1, emb2, emb3, init0_e):
    raise NotImplementedError("write your pallas kernel here")


# speedup vs baseline: 5.0322x; 5.0322x over previous
import functools

import jax
import jax.numpy as jnp
from jax import lax
from jax.experimental import pallas as pl
from jax.experimental.pallas import tpu as pltpu
from jax.experimental.pallas import tpu_sc as plsc

def _gelu(t):
    return 0.5 * t * (1.0 + lax.erf(t * 0.7071067811865476))


N = 10000
E = 320000
W = 128
H = 64

NC = 2
NS = 16
NW = NC * NS
CH = 80
ROWS = E // CH
PW_ROWS = ROWS // NW
K_FIRE = 5
OUTER = PW_ROWS // K_FIRE
GB = K_FIRE * CH

def _mesh():
    return plsc.VectorSubcoreMesh(core_axis_name="c", subcore_axis_name="s",
                                  num_cores=NC, num_subcores=NS)


def _k1_body(x_ref, wpre_ref, bpre_ref, w3_ref, b3_ref, w0_ref, b0_ref,
             y3_ref, z_ref):
    xx = jnp.dot(x_ref[...], wpre_ref[...], preferred_element_type=jnp.float32)
    xx = xx + bpre_ref[...]
    m = jnp.mean(xx, axis=-1, keepdims=True)
    v = jnp.mean(jnp.square(xx - m), axis=-1, keepdims=True)
    xx = (xx - m) / jnp.sqrt(v + 1e-5)
    y3 = jnp.dot(xx, w3_ref[...], preferred_element_type=jnp.float32)
    y3_ref[...] = y3 + b3_ref[...]
    z = jnp.dot(xx, w0_ref[...], preferred_element_type=jnp.float32)
    z_ref[...] = _gelu(z + b0_ref[...])


def _k1(x, w_pre, b_pre, w3, b3, w0, b0):
    R = 1000
    full = pl.BlockSpec((W, W), lambda i: (0, 0))
    row = pl.BlockSpec((1, W), lambda i: (0, 0))
    return pl.pallas_call(
        _k1_body,
        grid=(N // R,),
        in_specs=[pl.BlockSpec((R, W), lambda i: (i, 0)),
                  full, row, full, row, full, row],
        out_specs=[pl.BlockSpec((R, W), lambda i: (i, 0)),
                   pl.BlockSpec((R, W), lambda i: (i, 0))],
        out_shape=[jax.ShapeDtypeStruct((N, W), jnp.float32),
                   jax.ShapeDtypeStruct((N, W), jnp.float32)],
    )(x, w_pre, b_pre, w3, b3, w0, b0)


def _s1_body(y3_hbm, src_hbm, g_hbm, idx_v, gbuf, sem):
    cid = lax.axis_index("c")
    sid = lax.axis_index("s")
    wid = sid * NC + cid
    row0 = wid * PW_ROWS
    pltpu.sync_copy(src_hbm.at[wid], idx_v)

    def step(j, carry):
        cps = []
        for u in range(K_FIRE):
            cp = pltpu.async_copy(
                y3_hbm.at[idx_v.at[j * K_FIRE + u]],
                gbuf.at[pl.ds(u * CH, CH)], sem)
            cps.append(cp)
        for cp in cps:
            cp.wait()
        eoff = (row0 + j * K_FIRE) * CH
        pltpu.sync_copy(gbuf, g_hbm.at[pl.ds(eoff, GB)])
        return carry

    lax.fori_loop(0, OUTER, step, 0)


def _s1(y3i, src):
    run = functools.partial(
        pl.kernel, mesh=_mesh(),
        out_type=jax.ShapeDtypeStruct((E, W), jnp.int32),
        scratch_types=[pltpu.VMEM((PW_ROWS, CH), jnp.int32),
                       pltpu.VMEM((GB, W), jnp.int32),
                       pltpu.SemaphoreType.DMA],
    )(_s1_body)
    return run(y3i, src)


def _k2_body(g_ref, ee_ref, ea_ref, wr2_ref, tcat_ref, c_ref, msg_ref):
    B = g_ref.shape[0]
    gf = g_ref[...].astype(jnp.float32)
    acc = jnp.dot(ee_ref[...], wr2_ref[...], preferred_element_type=jnp.float32)
    idx = ea_ref[...]
    cols = lax.broadcasted_iota(jnp.int32, (B, 12), 1)
    tgt = cols % 3
    kk = cols // 3
    sel = jnp.where(kk == 0, idx[:, 0:1],
                    jnp.where(kk == 1, idx[:, 1:2],
                              jnp.where(kk == 2, idx[:, 2:3], idx[:, 3:4])))
    oh = (sel == tgt).astype(jnp.float32)
    acc = acc + jnp.dot(oh, tcat_ref[...], preferred_element_type=jnp.float32)
    v = gf + acc
    msg_ref[...] = _gelu(v) * c_ref[...]


def _k2(g_bf, ee, ea, wr2s, tcat, cvec):
    B = 2560
    return pl.pallas_call(
        _k2_body,
        grid=(E // B,),
        in_specs=[pl.BlockSpec((B, W), lambda i: (i, 0)),
                  pl.BlockSpec((B, H), lambda i: (i, 0)),
                  pl.BlockSpec((B, 4), lambda i: (i, 0)),
                  pl.BlockSpec((H, W), lambda i: (0, 0)),
                  pl.BlockSpec((12, W), lambda i: (0, 0)),
                  pl.BlockSpec((1, W), lambda i: (0, 0))],
        out_specs=pl.BlockSpec((B, W), lambda i: (i, 0)),
        out_shape=jax.ShapeDtypeStruct((E, W), jnp.float32),
    )(g_bf, ee, ea, wr2s, tcat, cvec)


def _s2_body(msg_hbm, dst_hbm, agg_hbm, idx_v, mbuf, zbuf, agg_s):
    cid = lax.axis_index("c")
    sid = lax.axis_index("s")
    wid = sid * NC + cid
    row0 = wid * PW_ROWS

    for r in range(16):
        for l in range(8):
            zbuf[r, pl.ds(l * 16, 16)] = jnp.zeros((16,), jnp.float32)
    stripe0 = sid * 624
    nz = jnp.where(sid == NS - 1, 40, 39)

    def zstep(t, carry):
        pltpu.sync_copy(zbuf, agg_s.at[pl.ds(stripe0 + t * 16, 16)])
        return carry

    lax.fori_loop(0, nz, zstep, 0)
    plsc.subcore_barrier()

    pltpu.sync_copy(dst_hbm.at[wid], idx_v)

    def step(j, carry):
        eoff = (row0 + j) * CH
        pltpu.sync_copy(msg_hbm.at[pl.ds(eoff, CH)], mbuf)
        pltpu.sync_copy(mbuf, agg_s.at[idx_v.at[j]], add=True)
        return carry

    lax.fori_loop(0, PW_ROWS, step, 0)
    plsc.subcore_barrier()

    def dstep(t, carry):
        pltpu.sync_copy(agg_s.at[pl.ds(stripe0 + t * 16, 16)], zbuf)
        pltpu.sync_copy(zbuf, agg_hbm.at[cid, pl.ds(stripe0 + t * 16, 16)])
        return carry

    lax.fori_loop(0, nz, dstep, 0)


def _s2(msg, dst):
    run = functools.partial(
        pl.kernel, mesh=_mesh(),
        out_type=jax.ShapeDtypeStruct((NC, N, W), jnp.float32),
        scratch_types=[pltpu.VMEM((PW_ROWS, CH), jnp.int32),
                       pltpu.VMEM((CH, W), jnp.float32),
                       pltpu.VMEM((16, W), jnp.float32),
                       pltpu.VMEM_SHARED((N, W), jnp.float32)],
    )(_s2_body)
    return run(msg, dst)


def _k3_body(x_ref, z_ref, a0_ref, a1_ref, wp_ref, bp_ref, o0_ref, o1_ref):
    xx2 = z_ref[...] + a0_ref[...] + a1_ref[...]
    out = jnp.dot(xx2, wp_ref[...], preferred_element_type=jnp.float32)
    out = out + bp_ref[...]
    o1_ref[...] = out
    o0_ref[...] = x_ref[...] + out


def _k3(x, z, a0, a1, w_post, b_post):
    R = 1000
    blk = pl.BlockSpec((R, W), lambda i: (i, 0))
    return pl.pallas_call(
        _k3_body,
        grid=(N // R,),
        in_specs=[blk, blk, blk, blk,
                  pl.BlockSpec((W, W), lambda i: (0, 0)),
                  pl.BlockSpec((1, W), lambda i: (0, 0))],
        out_specs=[blk, blk],
        out_shape=[jax.ShapeDtypeStruct((N, W), jnp.float32),
                   jax.ShapeDtypeStruct((N, W), jnp.float32)],
    )(x, z, a0, a1, w_post, b_post)


def kernel(x, edge_index, edge_attr, edge_embed,
           W_pre, b_pre,
           W_msg0, b_msg0, W_msg1, b_msg1, W_msg2, b_msg2, W_msg3, b_msg3,
           W_r0, b_r0, W_r1, b_r1, W_r2, b_r2,
           W_post, b_post, init0,
           emb0, emb1, emb2, emb3, init0_e):
    s = jnp.exp(init0[-1])
    c = jnp.exp(init0[0])
    ex = jnp.exp(init0_e)
    xw = ex / jnp.sqrt(jnp.sum(ex))

    wr2s = W_r2 * (s * 0.5)
    tsrc = jnp.concatenate([emb0[0:3] * xw[0], emb1[0:3] * xw[1],
                            emb2[0:3] * xw[2], emb3[0:3] * xw[3]], axis=0)
    tcat = tsrc @ wr2s
    bias3 = (b_msg3 + s * b_r2).reshape(1, W)
    cvec = jnp.full((1, W), c, jnp.float32)

    src = edge_index[0, 0].reshape(NW, PW_ROWS, CH)
    dst = edge_index[0, 1].reshape(NW, PW_ROWS, CH)
    ea = edge_attr[0]
    ee = edge_embed[0]

    y3, z = _k1(x, W_pre, b_pre.reshape(1, W), W_msg3, bias3,
                W_msg0, b_msg0.reshape(1, W))
    y3i = lax.bitcast_convert_type(y3, jnp.int32)

    g_i32 = _s1(y3i, src)
    g_f = lax.bitcast_convert_type(g_i32, jnp.float32)

    msg = _k2(g_f, ee, ea, wr2s, tcat, cvec)

    agg2 = _s2(msg, dst)

    o0, o1 = _k3(x, z, agg2[0], agg2[1], W_post, b_post.reshape(1, W))
    return (o0, o1, edge_embed)


# speedup vs baseline: 5.7848x; 1.1496x over previous
import functools

import jax
import jax.numpy as jnp
from jax import lax
from jax.experimental import pallas as pl
from jax.experimental.pallas import tpu as pltpu
from jax.experimental.pallas import tpu_sc as plsc

def _gelu(t):
    return 0.5 * t * (1.0 + lax.erf(t * 0.7071067811865476))


N = 10000
E = 320000
W = 128
H = 64

NC = 2
NS = 16
NW = NC * NS
CH = 80
ROWS = E // CH
PW_ROWS = ROWS // NW
K_FIRE = 5
OUTER = PW_ROWS // K_FIRE
GB = K_FIRE * CH

def _mesh():
    return plsc.VectorSubcoreMesh(core_axis_name="c", subcore_axis_name="s",
                                  num_cores=NC, num_subcores=NS)


def _k1_body(x_ref, wpre_ref, bpre_ref, w3_ref, b3_ref, w0_ref, b0_ref,
             y3_ref, z_ref):
    xx = jnp.dot(x_ref[...], wpre_ref[...], preferred_element_type=jnp.float32)
    xx = xx + bpre_ref[...]
    m = jnp.mean(xx, axis=-1, keepdims=True)
    v = jnp.mean(jnp.square(xx - m), axis=-1, keepdims=True)
    xx = (xx - m) / jnp.sqrt(v + 1e-5)
    y3 = jnp.dot(xx, w3_ref[...], preferred_element_type=jnp.float32)
    y3_ref[...] = y3 + b3_ref[...]
    z = jnp.dot(xx, w0_ref[...], preferred_element_type=jnp.float32)
    z_ref[...] = _gelu(z + b0_ref[...])


def _k1(x, w_pre, b_pre, w3, b3, w0, b0):
    R = 1000
    full = pl.BlockSpec((W, W), lambda i: (0, 0))
    row = pl.BlockSpec((1, W), lambda i: (0, 0))
    return pl.pallas_call(
        _k1_body,
        grid=(N // R,),
        in_specs=[pl.BlockSpec((R, W), lambda i: (i, 0)),
                  full, row, full, row, full, row],
        out_specs=[pl.BlockSpec((R, W), lambda i: (i, 0)),
                   pl.BlockSpec((R, W), lambda i: (i, 0))],
        out_shape=[jax.ShapeDtypeStruct((N, W), jnp.float32),
                   jax.ShapeDtypeStruct((N, W), jnp.float32)],
    )(x, w_pre, b_pre, w3, b3, w0, b0)


def _s1_body(y3_hbm, src_hbm, g_hbm, idx_v, gbuf, sem):
    cid = lax.axis_index("c")
    sid = lax.axis_index("s")
    wid = sid * NC + cid
    row0 = wid * PW_ROWS
    pltpu.sync_copy(src_hbm.at[wid], idx_v)

    def step(j, carry):
        cps = []
        for u in range(K_FIRE):
            cp = pltpu.async_copy(
                y3_hbm.at[idx_v.at[j * K_FIRE + u]],
                gbuf.at[pl.ds(u * CH, CH)], sem)
            cps.append(cp)
        for cp in cps:
            cp.wait()
        eoff = (row0 + j * K_FIRE) * CH
        pltpu.sync_copy(gbuf, g_hbm.at[pl.ds(eoff, GB)])
        return carry

    lax.fori_loop(0, OUTER, step, 0)


def _s1(y3i, src):
    run = functools.partial(
        pl.kernel, mesh=_mesh(),
        out_type=jax.ShapeDtypeStruct((E, W), jnp.float32),
        scratch_types=[pltpu.VMEM((PW_ROWS, CH), jnp.int32),
                       pltpu.VMEM((GB, W), jnp.float32),
                       pltpu.SemaphoreType.DMA],
    )(_s1_body)
    return run(y3i, src)


def _k2_body(g_ref, ee_ref, ea_ref, wr2_ref, tcat_ref, c_ref, msg_ref):
    B = g_ref.shape[0]
    gf = g_ref[...].astype(jnp.float32)
    acc = jnp.dot(ee_ref[...], wr2_ref[...], preferred_element_type=jnp.float32)
    idx = ea_ref[...]
    cols = lax.broadcasted_iota(jnp.int32, (B, 12), 1)
    tgt = cols % 3
    kk = cols // 3
    sel = jnp.where(kk == 0, idx[:, 0:1],
                    jnp.where(kk == 1, idx[:, 1:2],
                              jnp.where(kk == 2, idx[:, 2:3], idx[:, 3:4])))
    oh = (sel == tgt).astype(jnp.float32)
    acc = acc + jnp.dot(oh, tcat_ref[...], preferred_element_type=jnp.float32)
    v = gf + acc
    msg_ref[...] = _gelu(v) * c_ref[...]


def _k2(g_bf, ee, ea, wr2s, tcat, cvec):
    B = 2560
    return pl.pallas_call(
        _k2_body,
        grid=(E // B,),
        in_specs=[pl.BlockSpec((B, W), lambda i: (i, 0)),
                  pl.BlockSpec((B, H), lambda i: (i, 0)),
                  pl.BlockSpec((B, 4), lambda i: (i, 0)),
                  pl.BlockSpec((H, W), lambda i: (0, 0)),
                  pl.BlockSpec((12, W), lambda i: (0, 0)),
                  pl.BlockSpec((1, W), lambda i: (0, 0))],
        out_specs=pl.BlockSpec((B, W), lambda i: (i, 0)),
        out_shape=jax.ShapeDtypeStruct((E, W), jnp.float32),
    )(g_bf, ee, ea, wr2s, tcat, cvec)


def _s2_body(msg_hbm, dst_hbm, agg_hbm, idx_v, mbuf, zbuf, agg_s):
    cid = lax.axis_index("c")
    sid = lax.axis_index("s")
    wid = sid * NC + cid
    row0 = wid * PW_ROWS

    for r in range(16):
        for l in range(8):
            zbuf[r, pl.ds(l * 16, 16)] = jnp.zeros((16,), jnp.float32)
    stripe0 = sid * 624
    nz = jnp.where(sid == NS - 1, 40, 39)

    def zstep(t, carry):
        pltpu.sync_copy(zbuf, agg_s.at[pl.ds(stripe0 + t * 16, 16)])
        return carry

    lax.fori_loop(0, nz, zstep, 0)
    plsc.subcore_barrier()

    pltpu.sync_copy(dst_hbm.at[wid], idx_v)

    def step(j, carry):
        eoff = (row0 + j) * CH
        pltpu.sync_copy(msg_hbm.at[pl.ds(eoff, CH)], mbuf)
        pltpu.sync_copy(mbuf, agg_s.at[idx_v.at[j]], add=True)
        return carry

    lax.fori_loop(0, PW_ROWS, step, 0)
    plsc.subcore_barrier()

    def dstep(t, carry):
        pltpu.sync_copy(agg_s.at[pl.ds(stripe0 + t * 16, 16)], zbuf)
        pltpu.sync_copy(zbuf, agg_hbm.at[cid, pl.ds(stripe0 + t * 16, 16)])
        return carry

    lax.fori_loop(0, nz, dstep, 0)


def _s2(msg, dst):
    run = functools.partial(
        pl.kernel, mesh=_mesh(),
        out_type=jax.ShapeDtypeStruct((NC, N, W), jnp.float32),
        scratch_types=[pltpu.VMEM((PW_ROWS, CH), jnp.int32),
                       pltpu.VMEM((CH, W), jnp.float32),
                       pltpu.VMEM((16, W), jnp.float32),
                       pltpu.VMEM_SHARED((N, W), jnp.float32)],
    )(_s2_body)
    return run(msg, dst)


def _k3_body(x_ref, z_ref, a0_ref, a1_ref, wp_ref, bp_ref, o0_ref, o1_ref):
    xx2 = z_ref[...] + a0_ref[...] + a1_ref[...]
    out = jnp.dot(xx2, wp_ref[...], preferred_element_type=jnp.float32)
    out = out + bp_ref[...]
    o1_ref[...] = out
    o0_ref[...] = x_ref[...] + out


def _k3(x, z, a0, a1, w_post, b_post):
    R = 1000
    blk = pl.BlockSpec((R, W), lambda i: (i, 0))
    return pl.pallas_call(
        _k3_body,
        grid=(N // R,),
        in_specs=[blk, blk, blk, blk,
                  pl.BlockSpec((W, W), lambda i: (0, 0)),
                  pl.BlockSpec((1, W), lambda i: (0, 0))],
        out_specs=[blk, blk],
        out_shape=[jax.ShapeDtypeStruct((N, W), jnp.float32),
                   jax.ShapeDtypeStruct((N, W), jnp.float32)],
    )(x, z, a0, a1, w_post, b_post)


def kernel(x, edge_index, edge_attr, edge_embed,
           W_pre, b_pre,
           W_msg0, b_msg0, W_msg1, b_msg1, W_msg2, b_msg2, W_msg3, b_msg3,
           W_r0, b_r0, W_r1, b_r1, W_r2, b_r2,
           W_post, b_post, init0,
           emb0, emb1, emb2, emb3, init0_e):
    s = jnp.exp(init0[-1])
    c = jnp.exp(init0[0])
    ex = jnp.exp(init0_e)
    xw = ex / jnp.sqrt(jnp.sum(ex))

    wr2s = W_r2 * (s * 0.5)
    tsrc = jnp.concatenate([emb0[0:3] * xw[0], emb1[0:3] * xw[1],
                            emb2[0:3] * xw[2], emb3[0:3] * xw[3]], axis=0)
    tcat = tsrc @ wr2s
    bias3 = (b_msg3 + s * b_r2).reshape(1, W)
    cvec = jnp.full((1, W), c, jnp.float32)

    src = edge_index[0, 0].reshape(NW, PW_ROWS, CH)
    dst = edge_index[0, 1].reshape(NW, PW_ROWS, CH)
    ea = edge_attr[0]
    ee = edge_embed[0]

    y3, z = _k1(x, W_pre, b_pre.reshape(1, W), W_msg3, bias3,
                W_msg0, b_msg0.reshape(1, W))

    g_f = _s1(y3, src)

    msg = _k2(g_f, ee, ea, wr2s, tcat, cvec)

    agg2 = _s2(msg, dst)

    o0, o1 = _k3(x, z, agg2[0], agg2[1], W_post, b_post.reshape(1, W))
    return (o0, o1, edge_embed)
